# baseline (speedup 1.0000x reference)
import jax
import jax.numpy as jnp
from jax.experimental import pallas as pl

M, D, B = 262144, 64, 16384


def _copy_body(x_ref, o_ref):
    o_ref[...] = x_ref[...]


_tc_copy = pl.pallas_call(
    _copy_body,
    grid=(32,),
    in_specs=[pl.BlockSpec((M // 32, D), lambda i: (i, 0))],
    out_specs=pl.BlockSpec((M // 32, D), lambda i: (i, 0)),
    out_shape=jax.ShapeDtypeStruct((M, D), jnp.float32),
)


def kernel(mem, val, idx):
    out0 = _tc_copy(mem)
    return out0.at[idx].set(val)


# speedup vs baseline: 7.0835x; 7.0835x over previous
import jax
import jax.numpy as jnp
from jax import lax
from jax.experimental import pallas as pl
from jax.experimental.pallas import tpu as pltpu
from jax.experimental.pallas import tpu_sc as plsc

M, D, B = 262144, 64, 16384
DW = 128
NC, NS, L = 2, 16, 16
NW = NC * NS
SEG = M // NW
PERW = B // NW
CHUNK = 128
NCHUNK = PERW // CHUNK
POS_BITS = 14
POS_MASK = (1 << POS_BITS) - 1

_mesh = plsc.VectorSubcoreMesh(core_axis_name="c", subcore_axis_name="s")


def _wid():
    return lax.axis_index("s") * NC + lax.axis_index("c")


def _lane_gather(x, perm):
    dnums = lax.GatherDimensionNumbers(
        offset_dims=(), collapsed_slice_dims=(0,), start_index_map=(0,))
    return lax.gather(x, perm[:, None], dnums, (1,),
                      mode=lax.GatherScatterMode.PROMISE_IN_BOUNDS)


def _widen_body(x_ref, o_ref):
    x = x_ref[...]
    o_ref[...] = jnp.concatenate([x, jnp.zeros_like(x)], axis=1)


def _widen(n_rows, grid):
    blk = n_rows // grid
    return pl.pallas_call(
        _widen_body,
        grid=(grid,),
        in_specs=[pl.BlockSpec((blk, D), lambda i: (i, 0))],
        out_specs=pl.BlockSpec((blk, DW), lambda i: (i, 0)),
        out_shape=jax.ShapeDtypeStruct((n_rows, DW), jnp.float32),
    )


_widen_mem = _widen(M, 32)
_widen_val = _widen(B, 4)


def _dedup_body(idx_hbm, w_hbm, idx_v, wseg_v):
    wid = _wid()
    lo = wid * SEG
    pltpu.sync_copy(idx_hbm, idx_v)

    lane = lax.iota(jnp.int32, L)
    nxt_perm = jnp.minimum(lane + 1, L - 1)
    is_last_lane = lane == (L - 1)

    def body(g, _):
        v = idx_v[pl.ds(g * L, L)]
        pos = g * L + lane
        c = lax.shift_left(v, POS_BITS) | pos
        cs, _ = plsc.sort_key_val(c, c)
        row = lax.shift_right_logical(cs, POS_BITS)
        p = cs & POS_MASK
        nxt_row = _lane_gather(row, nxt_perm)
        keep = (row != nxt_row) | is_last_lane
        in_seg = (row >= lo) & (row < lo + SEG)
        m = keep & in_seg
        local = jnp.where(m, row - lo, 0)
        plsc.store_scatter(wseg_v, [local], p, mask=m)
        return 0

    lax.fori_loop(0, B // L, body, 0)
    pltpu.sync_copy(wseg_v, w_hbm.at[pl.ds(lo, SEG)])


_sc_dedup = pl.kernel(
    _dedup_body,
    out_type=jax.ShapeDtypeStruct((M,), jnp.int32),
    mesh=_mesh,
    scratch_types=[
        pltpu.VMEM((B,), jnp.int32),
        pltpu.VMEM((SEG,), jnp.int32),
    ],
    compiler_params=pltpu.CompilerParams(needs_layout_passes=False),
)


def _scatter_body(out_ref, val_hbm, idx_hbm, w_hbm, idx_v, win_v, rows_v, sem):
    wid = _wid()
    base = wid * PERW
    for j in range(NCHUNK):
        pltpu.sync_copy(idx_hbm.at[pl.ds(base + j * CHUNK, CHUNK)], idx_v.at[j])
    for j in range(NCHUNK):
        pltpu.async_copy(w_hbm.at[idx_v.at[j]], win_v.at[j], sem).wait()
    for j in range(NCHUNK):
        pltpu.async_copy(val_hbm.at[win_v.at[j]], rows_v.at[j], sem).wait()
    for j in range(NCHUNK):
        pltpu.async_copy(rows_v.at[j], out_ref.at[idx_v.at[j]], sem).wait()


_sc_scatter = pl.kernel(
    _scatter_body,
    out_type=(),
    mesh=_mesh,
    scratch_types=[
        pltpu.VMEM((NCHUNK, CHUNK), jnp.int32),
        pltpu.VMEM((NCHUNK, CHUNK), jnp.int32),
        pltpu.VMEM((NCHUNK, CHUNK, DW), jnp.float32),
        pltpu.SemaphoreType.DMA,
    ],
)


def kernel(mem, val, idx):
    idx32 = idx.astype(jnp.int32)
    out128 = _widen_mem(mem)
    val128 = _widen_val(val)
    w = _sc_dedup(idx32)
    out_ref = jax.new_ref(out128)
    _sc_scatter(out_ref, val128, idx32, w)
    return jax.freeze(out_ref)[:, :D]
